# Initial kernel scaffold; baseline (speedup 1.0000x reference)
import jax
import jax.numpy as jnp
from jax.experimental import pallas as pl


def kernel(x, edge_index, W1, b1, W2, b2, gamma, beta):
    raise NotImplementedError("write your pallas kernel here")


# speedup vs baseline: 5.5083x; 5.5083x over previous
import functools

import jax
import jax.numpy as jnp
from jax import lax
from jax.experimental import pallas as pl
from jax.experimental.pallas import tpu as pltpu
from jax.experimental.pallas import tpu_sc as plsc

N = 10000
E = 320000
D = 128
BN_EPS = 1e-5

NC = 2
NS = 16
NW = NC * NS
EPW = E // NW
CH = 80
NCHUNK = EPW // CH
SPAN = 624
NSP = 6
ZR = 104
TAIL = 16
TAIL0 = NS * SPAN
LPR = D // 16


def _sc_aggregate(x, src, dst):
  mesh = plsc.VectorSubcoreMesh(
      core_axis_name="c", subcore_axis_name="s",
      num_cores=NC, num_subcores=NS)

  @functools.partial(
      pl.kernel,
      out_type=jax.ShapeDtypeStruct((NC, N, D), jnp.float32),
      mesh=mesh,
      scratch_types=[
          pltpu.VMEM((CH,), jnp.int32),
          pltpu.VMEM((CH,), jnp.int32),
          pltpu.VMEM((CH, D), jnp.float32),
          pltpu.VMEM((ZR, D), jnp.float32),
          pltpu.SemaphoreType.DMA,
          pltpu.VMEM_SHARED((N, D), jnp.float32),
      ],
  )
  def agg_kernel(x_hbm, src_hbm, dst_hbm, out_hbm, src_v, dst_v, rows_v,
                 stage_v, sem, acc_sh):
    c = lax.axis_index("c")
    s = lax.axis_index("s")
    wid = c * NS + s

    zeros16 = jnp.zeros((16,), jnp.float32)

    def zbody(i, carry):
      stage_v[i // LPR, pl.ds((i % LPR) * 16, 16)] = zeros16
      return carry

    lax.fori_loop(0, ZR * LPR, zbody, 0)
    for k in range(NSP):
      pltpu.sync_copy(stage_v, acc_sh.at[pl.ds(s * SPAN + k * ZR, ZR)])

    @pl.when(s == 0)
    def _():
      pltpu.sync_copy(stage_v.at[pl.ds(0, TAIL)], acc_sh.at[pl.ds(TAIL0, TAIL)])

    plsc.subcore_barrier()

    def chunk_body(i, carry):
      base = wid * EPW + i * CH
      pltpu.sync_copy(src_hbm.at[pl.ds(base, CH)], src_v)
      pltpu.sync_copy(dst_hbm.at[pl.ds(base, CH)], dst_v)
      pltpu.async_copy(x_hbm.at[src_v], rows_v, sem).wait()
      pltpu.sync_copy(rows_v, acc_sh.at[dst_v], add=True)
      return carry

    lax.fori_loop(0, NCHUNK, chunk_body, 0)
    plsc.subcore_barrier()

    for k in range(NSP):
      row0 = s * SPAN + k * ZR
      pltpu.sync_copy(acc_sh.at[pl.ds(row0, ZR)], stage_v)
      pltpu.sync_copy(stage_v, out_hbm.at[c, pl.ds(row0, ZR)])

    @pl.when(s == 0)
    def _():
      pltpu.sync_copy(acc_sh.at[pl.ds(TAIL0, TAIL)], stage_v.at[pl.ds(0, TAIL)])
      pltpu.sync_copy(stage_v.at[pl.ds(0, TAIL)],
                      out_hbm.at[c, pl.ds(TAIL0, TAIL)])

  return agg_kernel(x, src, dst)


def _tc_mlp(x, parts, W1, b1, W2, b2, gamma, beta):
  def body(x_ref, p_ref, w1_ref, b1_ref, w2_ref, b2_ref, g_ref, bt_ref,
           o_ref):
    xx = x_ref[...]
    h = xx + p_ref[0] + p_ref[1]
    h = jnp.dot(h, w1_ref[...], preferred_element_type=jnp.float32)
    h = jnp.maximum(h + b1_ref[...], 0.0)
    h = jnp.dot(h, w2_ref[...], preferred_element_type=jnp.float32)
    h = h + b2_ref[...]
    mean = jnp.mean(h, axis=0, keepdims=True)
    cen = h - mean
    var = jnp.mean(cen * cen, axis=0, keepdims=True)
    h = cen * lax.rsqrt(var + BN_EPS) * g_ref[...] + bt_ref[...]
    o_ref[...] = xx + jnp.maximum(h, 0.0)

  return pl.pallas_call(
      body,
      out_shape=jax.ShapeDtypeStruct((N, D), jnp.float32),
  )(x, parts, W1, b1.reshape(1, D), W2, b2.reshape(1, D),
    gamma.reshape(1, D), beta.reshape(1, D))


def kernel(x, edge_index, W1, b1, W2, b2, gamma, beta):
  parts = _sc_aggregate(x, edge_index[0], edge_index[1])
  return _tc_mlp(x, parts, W1, b1, W2, b2, gamma, beta)
